# Initial kernel scaffold; baseline (speedup 1.0000x reference)
import jax
import jax.numpy as jnp
from jax.experimental import pallas as pl


def kernel(x, edges, att_proj_W, att_proj_b, att_map_W, proj_att_W, proj_att_b, proj_noatt_W, proj_noatt_b, bn_gamma, bn_beta, y_proj_W, y_proj_b):
    raise NotImplementedError("write your pallas kernel here")


# speedup vs baseline: 2.0993x; 2.0993x over previous
import functools

import jax
import jax.numpy as jnp
from jax import lax
from jax.experimental import pallas as pl
from jax.experimental.pallas import tpu as pltpu
from jax.experimental.pallas import tpu_sc as plsc

F = 128
NC = 2
NS = 16
NW = NC * NS


def _make_gather(N, E):
    EW = E // NW
    CHUNK = 80
    NCH = EW // CHUNK

    mesh = plsc.VectorSubcoreMesh(core_axis_name="c", subcore_axis_name="s")

    @functools.partial(
        pl.kernel,
        out_type=[jax.ShapeDtypeStruct((E, F), jnp.float32),
                  jax.ShapeDtypeStruct((E, F), jnp.float32)],
        mesh=mesh,
        compiler_params=pltpu.CompilerParams(needs_layout_passes=False),
        scratch_types=[
            pltpu.VMEM((NCH, CHUNK), jnp.int32),
            pltpu.VMEM((NCH, CHUNK), jnp.int32),
            pltpu.VMEM((CHUNK, F), jnp.float32),
            pltpu.VMEM((CHUNK, F), jnp.float32),
            pltpu.SemaphoreType.DMA,
            pltpu.SemaphoreType.DMA,
        ],
    )
    def gather_k(flat, src3, dst3, xi_out, xj_out, src_v, dst_v, bi, bj,
                 sem_i, sem_j):
        wid = lax.axis_index("c") * NS + lax.axis_index("s")
        base = wid * EW
        pltpu.sync_copy(src3.at[wid], src_v)
        pltpu.sync_copy(dst3.at[wid], dst_v)

        def body(j, _):
            cpi = pltpu.async_copy(flat.at[dst_v.at[j]], bi, sem_i)
            cpj = pltpu.async_copy(flat.at[src_v.at[j]], bj, sem_j)
            cpi.wait()
            cpj.wait()
            pltpu.sync_copy(bi, xi_out.at[pl.ds(base + j * CHUNK, CHUNK)])
            pltpu.sync_copy(bj, xj_out.at[pl.ds(base + j * CHUNK, CHUNK)])
            return 0

        lax.fori_loop(0, NCH, body, 0)

    def run(flat, src, dst):
        src3 = src.reshape(NW, NCH, CHUNK)
        dst3 = dst.reshape(NW, NCH, CHUNK)
        return gather_k(flat, src3, dst3)

    return run


def _make_segmax(N, E):
    NP = ((N + NW * 8 - 1) // (NW * 8)) * (NW * 8)
    SPW = NP // NW
    CH = 2000
    NCHS = E // CH
    VPC = CH // 16
    CAP = 13056

    mesh = plsc.VectorSubcoreMesh(core_axis_name="c", subcore_axis_name="s")

    @functools.partial(
        pl.kernel,
        out_type=jax.ShapeDtypeStruct((NP,), jnp.float32),
        mesh=mesh,
        compiler_params=pltpu.CompilerParams(needs_layout_passes=False),
        scratch_types=[
            pltpu.VMEM((SPW, 16), jnp.float32),
            pltpu.VMEM((CH,), jnp.int32),
            pltpu.VMEM((CH,), jnp.float32),
            pltpu.VMEM((CAP,), jnp.float32),
            pltpu.VMEM((CAP,), jnp.int32),
            pltpu.VMEM((SPW,), jnp.float32),
        ],
    )
    def segmax_k(dst_h, alpha_h, m_out, accm, dstb, alphab, cal, cls, mbuf):
        wid = lax.axis_index("c") * NS + lax.axis_index("s")
        seg_base = wid * SPW
        lanes = lax.iota(jnp.int32, 16)

        def init(i, _):
            accm[i, pl.ds(0, 16)] = jnp.full((16,), -1e30, jnp.float32)
            return 0

        lax.fori_loop(0, SPW, init, 0)

        def scan_chunk(ci, cnt):
            cnt = jnp.minimum(cnt, jnp.int32(CAP - CH - 16))
            pltpu.sync_copy(dst_h.at[pl.ds(ci * CH, CH)], dstb)
            pltpu.sync_copy(alpha_h.at[pl.ds(ci * CH, CH)], alphab)

            def vreg(i, cnt):
                d = dstb[pl.ds(i * 16, 16)]
                a = alphab[pl.ds(i * 16, 16)]
                local = d - seg_base
                inb = (local >= 0) & (local < SPW)
                pc = plsc.cumsum(jnp.where(inb, 1, 0))
                addr = cnt + pc - 1
                plsc.store_scatter(cal, [addr], a, mask=inb)
                plsc.store_scatter(cls, [addr], local, mask=inb)
                return cnt + pc[15]

            return lax.fori_loop(0, VPC, vreg, cnt)

        cnt = lax.fori_loop(0, NCHS, scan_chunk, jnp.int32(0))

        def edge(r, _):
            av = plsc.load_gather(cal, [jnp.full((16,), r, jnp.int32)])
            dl = plsc.load_gather(cls, [jnp.full((16,), r, jnp.int32)])[0]
            accm[dl, pl.ds(0, 16)] = jnp.maximum(accm[dl, pl.ds(0, 16)], av)
            return 0

        lax.fori_loop(0, cnt, edge, 0)

        def comb(i, _):
            s0 = i * 16
            mbuf[pl.ds(s0, 16)] = plsc.load_gather(
                accm, [s0 + lanes, jnp.zeros((16,), jnp.int32)])
            return 0

        lax.fori_loop(0, SPW // 16, comb, 0)
        pltpu.sync_copy(mbuf, m_out.at[pl.ds(seg_base, SPW)])

    return segmax_k


def _make_gather_sums(N, E):
    EW = E // NW
    VW = EW // 16
    mesh = plsc.VectorSubcoreMesh(core_axis_name="c", subcore_axis_name="s")

    @functools.partial(
        pl.kernel,
        out_type=jax.ShapeDtypeStruct((E,), jnp.float32),
        mesh=mesh,
        compiler_params=pltpu.CompilerParams(needs_layout_passes=False),
        scratch_types=[
            pltpu.VMEM((N,), jnp.float32),
            pltpu.VMEM((EW,), jnp.int32),
            pltpu.VMEM((EW,), jnp.float32),
        ],
    )
    def gsum_k(sums_h, dst_h, sd_out, sums_v, dstw, sd_buf):
        wid = lax.axis_index("c") * NS + lax.axis_index("s")
        ebase = wid * EW
        pltpu.sync_copy(sums_h, sums_v)
        pltpu.sync_copy(dst_h.at[pl.ds(ebase, EW)], dstw)

        def vreg(i, _):
            d = dstw[pl.ds(i * 16, 16)]
            sd_buf[pl.ds(i * 16, 16)] = plsc.load_gather(sums_v, [d])
            return 0

        lax.fori_loop(0, VW, vreg, 0)
        pltpu.sync_copy(sd_buf, sd_out.at[pl.ds(ebase, EW)])

    return gsum_k


def _make_weighted(N, E):
    NP = ((N + NW * 8 - 1) // (NW * 8)) * (NW * 8)
    SPW = NP // NW
    CH = 2000
    NCHS = E // CH
    VPC = CH // 16
    CAP = 13056
    GB = 64

    mesh = plsc.VectorSubcoreMesh(core_axis_name="c", subcore_axis_name="s")

    @functools.partial(
        pl.kernel,
        out_type=jax.ShapeDtypeStruct((NP, F), jnp.float32),
        mesh=mesh,
        compiler_params=pltpu.CompilerParams(needs_layout_passes=False),
        scratch_types=[
            pltpu.VMEM((SPW, F), jnp.float32),
            pltpu.VMEM((CH,), jnp.int32),
            pltpu.VMEM((CAP,), jnp.int32),
            pltpu.VMEM((CAP,), jnp.int32),
            pltpu.VMEM((GB, F), jnp.float32),
            pltpu.VMEM((GB + 16,), jnp.float32),
            pltpu.SemaphoreType.DMA,
            pltpu.SemaphoreType.DMA,
        ],
    )
    def wk(dst_h, attn_h, xj_h, zeros_h, w_out, acc, dstb, eids, ldst,
           rows, attnb, sem_r, sem_a):
        wid = lax.axis_index("c") * NS + lax.axis_index("s")
        seg_base = wid * SPW
        lanes = lax.iota(jnp.int32, 16)

        pltpu.sync_copy(zeros_h.at[pl.ds(seg_base, SPW)], acc)

        def scan_chunk(ci, cnt):
            cnt = jnp.minimum(cnt, jnp.int32(CAP - CH - 16))
            pltpu.sync_copy(dst_h.at[pl.ds(ci * CH, CH)], dstb)

            def vreg(i, cnt):
                d = dstb[pl.ds(i * 16, 16)]
                local = d - seg_base
                inb = (local >= 0) & (local < SPW)
                eid = (ci * CH + i * 16) + lanes
                pc = plsc.cumsum(jnp.where(inb, 1, 0))
                addr = cnt + pc - 1
                plsc.store_scatter(eids, [addr], eid, mask=inb)
                plsc.store_scatter(ldst, [addr], local, mask=inb)
                return cnt + pc[15]

            return lax.fori_loop(0, VPC, vreg, cnt)

        cnt = lax.fori_loop(0, NCHS, scan_chunk, jnp.int32(0))

        for i in range(GB // 16 + 1):
            plsc.store_scatter(eids, [cnt + i * 16 + lanes],
                               jnp.zeros((16,), jnp.int32))

        nb = lax.div(cnt + (GB - 1), jnp.int32(GB))

        def batch(bi, _):
            pos = bi * GB
            cp_r = pltpu.async_copy(xj_h.at[eids.at[pl.ds(pos, GB)]], rows,
                                    sem_r)
            cp_a = pltpu.async_copy(attn_h.at[eids.at[pl.ds(pos, GB)]],
                                    attnb.at[pl.ds(0, GB)], sem_a)
            cp_r.wait()
            cp_a.wait()
            n_in_batch = jnp.minimum(cnt - pos, GB)

            def row(r, _):
                ab = plsc.load_gather(attnb, [jnp.full((16,), r, jnp.int32)])
                dl = plsc.load_gather(ldst,
                                      [jnp.full((16,), pos + r, jnp.int32)])[0]
                for f in range(F // 16):
                    sl = pl.ds(f * 16, 16)
                    prod = ab * rows[r, sl]
                    acc[dl, sl] = acc[dl, sl] + prod
                return 0

            lax.fori_loop(0, n_in_batch, row, 0)
            return 0

        lax.fori_loop(0, nb, batch, 0)

        pltpu.sync_copy(acc, w_out.at[pl.ds(seg_base, SPW)])

    return wk


def _edge_alpha(Xi, Xj, Wa, ba, amap, E):
    BE = 640
    G = E // BE

    def body(xi_ref, xj_ref, w_ref, b_ref, a_ref, o_ref):
        p = xi_ref[...] * xj_ref[...]
        h = jnp.tanh(jnp.dot(p, w_ref[...]) + b_ref[...])
        row = lax.dot_general(a_ref[...], h, (((0,), (1,)), ((), ())))
        o_ref[0] = row

    out = pl.pallas_call(
        body,
        grid=(G,),
        in_specs=[
            pl.BlockSpec((BE, F), lambda i: (i, 0)),
            pl.BlockSpec((BE, F), lambda i: (i, 0)),
            pl.BlockSpec((F, F), lambda i: (0, 0)),
            pl.BlockSpec((1, F), lambda i: (0, 0)),
            pl.BlockSpec((F, 1), lambda i: (0, 0)),
        ],
        out_specs=pl.BlockSpec((1, 1, BE), lambda i: (i, 0, 0)),
        out_shape=jax.ShapeDtypeStruct((G, 1, BE), jnp.float32),
    )(Xi, Xj, Wa, ba.reshape(1, F), amap)
    return out.reshape(E)


def _ew_exp(alpha, md, E):
    R = 2500
    C = E // R

    def body(a_ref, m_ref, o_ref):
        o_ref[...] = jnp.exp(a_ref[...] - m_ref[...])

    out = pl.pallas_call(
        body,
        out_shape=jax.ShapeDtypeStruct((R, C), jnp.float32),
    )(alpha.reshape(R, C), md.reshape(R, C))
    return out.reshape(E)


def _ew_attn(ex, sd, E):
    R = 2500
    C = E // R

    def body(e_ref, s_ref, o_ref):
        o_ref[...] = e_ref[...] / (s_ref[...] + 1e-16)

    out = pl.pallas_call(
        body,
        out_shape=jax.ShapeDtypeStruct((R, C), jnp.float32),
    )(ex.reshape(R, C), sd.reshape(R, C))
    return out.reshape(E)


def _tail_matmul(weighted, flat, W1, b1, W2, b2, N):
    def body(w_ref, f_ref, w1_ref, b1_ref, w2_ref, b2_ref, o_ref):
        o_ref[...] = (jnp.dot(w_ref[...], w1_ref[...]) + b1_ref[...]
                      + jnp.dot(f_ref[...], w2_ref[...])) + b2_ref[...]

    return pl.pallas_call(
        body,
        out_shape=jax.ShapeDtypeStruct((N, F), jnp.float32),
    )(weighted, flat, W1, b1.reshape(1, F), W2, b2.reshape(1, F))


def _tail_norm(out, mean, var, gamma, beta, N):
    def body(o_ref, m_ref, v_ref, g_ref, be_ref, r_ref):
        r_ref[...] = ((o_ref[...] - m_ref[...]) * lax.rsqrt(v_ref[...] + 1e-5)
                      * g_ref[...] + be_ref[...])

    return pl.pallas_call(
        body,
        out_shape=jax.ShapeDtypeStruct((N, F), jnp.float32),
    )(out, mean.reshape(1, F), var.reshape(1, F), gamma.reshape(1, F),
      beta.reshape(1, F))


def _pool(out3, y, B, NPG, SEL_K):
    def body(o_ref, yr_ref, yc_ref, res_ref):
        og = o_ref[0]
        yrow = yr_ref[0]
        ycol = yc_ref[0]
        mi = lax.broadcasted_iota(jnp.int32, (NPG, 1), 0)
        ni = lax.broadcasted_iota(jnp.int32, (1, NPG), 1)
        C = ((ycol > yrow) | ((ycol == yrow) & (mi < ni))).astype(jnp.float32)
        rank = jnp.dot(jnp.ones((1, NPG), jnp.float32), C)
        r_col = lax.broadcasted_iota(jnp.int32, (SEL_K, 1), 0).astype(jnp.float32)
        O = (rank == r_col).astype(jnp.float32)
        Z = og * ycol
        res_ref[0] = jnp.dot(O, Z, precision=lax.Precision.HIGHEST)

    return pl.pallas_call(
        body,
        grid=(B,),
        in_specs=[
            pl.BlockSpec((1, NPG, F), lambda g: (g, 0, 0)),
            pl.BlockSpec((1, 1, NPG), lambda g: (g, 0, 0)),
            pl.BlockSpec((1, NPG, 1), lambda g: (g, 0, 0)),
        ],
        out_specs=pl.BlockSpec((1, SEL_K, F), lambda g: (g, 0, 0)),
        out_shape=jax.ShapeDtypeStruct((B, SEL_K, F), jnp.float32),
    )(out3, y.reshape(B, 1, NPG), y.reshape(B, NPG, 1))


def kernel(x, edges, att_proj_W, att_proj_b, att_map_W, proj_att_W,
           proj_att_b, proj_noatt_W, proj_noatt_b, bn_gamma, bn_beta,
           y_proj_W, y_proj_b):
    B, NPG, F_ = x.shape
    N = B * NPG
    E = edges.shape[1]
    SEL_K = max(int(NPG * 0.5), 1)

    flat = x.reshape(N, F_)
    src = edges[0]
    dst = edges[1]

    Xi, Xj = _make_gather(N, E)(flat, src, dst)
    alpha = _edge_alpha(Xi, Xj, att_proj_W, att_proj_b, att_map_W, E)
    m = _make_segmax(N, E)(dst, alpha)
    md = _make_gather_sums(m.shape[0], E)(m, dst)
    ex = _ew_exp(alpha, md, E)
    sums = jax.ops.segment_sum(ex, dst, num_segments=N)
    sd = _make_gather_sums(N, E)(sums, dst)
    attn = _ew_attn(ex, sd, E)
    NP = ((N + NW * 8 - 1) // (NW * 8)) * (NW * 8)
    zeros = jnp.zeros((NP, F_), jnp.float32)
    weighted = _make_weighted(N, E)(dst, attn, Xj, zeros)[:N]
    out = _tail_matmul(weighted, flat, proj_att_W, proj_att_b, proj_noatt_W,
                       proj_noatt_b, N)
    mean = out.mean(axis=0)
    var = out.var(axis=0)
    out_bn = (out - mean) / jnp.sqrt(var + 1e-5) * bn_gamma + bn_beta
    out3 = jax.nn.selu(out_bn).reshape(B, NPG, F_)
    y = jax.nn.sigmoid(out3 @ y_proj_W + y_proj_b)[..., 0]
    return _pool(out3, y, B, NPG, SEL_K)
